# baseline (speedup 1.0000x reference)
import functools

import jax
import jax.numpy as jnp
from jax import lax
from jax.experimental import pallas as pl
from jax.experimental.pallas import tpu as pltpu
from jax.experimental.pallas import tpu_sc as plsc

_NC = 2
_NS = 16
_NW = _NC * _NS


def kernel(patches):
    batch = patches.shape[0]
    x = jnp.transpose(patches, (0, 1, 2, 5, 3, 4))

    n_items = batch * 8 * 2 * 4
    ipw = n_items // _NW

    mesh = plsc.VectorSubcoreMesh(core_axis_name="c", subcore_axis_name="s")

    @functools.partial(
        pl.kernel,
        mesh=mesh,
        out_type=jax.ShapeDtypeStruct((batch, 3, 512, 512), jnp.float32),
        scratch_types=[
            pltpu.VMEM((4, 3, 16, 64), jnp.float32),
            pltpu.VMEM((4, 3, 16, 64), jnp.float32),
            pltpu.VMEM((3, 16, 256), jnp.float32),
            pltpu.VMEM((3, 16, 256), jnp.float32),
            pltpu.SemaphoreType.DMA,
            pltpu.SemaphoreType.DMA,
            pltpu.SemaphoreType.DMA,
            pltpu.SemaphoreType.DMA,
        ],
    )
    def unpatch(in_hbm, out_hbm,
                in_a, in_b, lin_a, lin_b,
                sin_a, sin_b, sout_a, sout_b):
        wid = lax.axis_index("s") * _NC + lax.axis_index("c")
        t0 = wid * ipw

        ins = [in_a, in_b]
        lins = [lin_a, lin_b]
        sins = [sin_a, sin_b]
        souts = [sout_a, sout_b]

        def coords(k):
            t = t0 + k
            b = t // 64
            r = t % 64
            j = r // 8
            q = r % 8
            return b, j, q // 4, q % 4

        def load(k):
            b, j, q, ph = coords(k)
            return [
                pltpu.async_copy(
                    in_hbm.at[b, j, pl.ds(4 * q, 4), :, pl.ds(ph * 16, 16), :],
                    ins[k % 2],
                    sins[k % 2],
                )
            ]

        def merge(k):
            bi, lin = ins[k % 2], lins[k % 2]

            def body(t, carry):
                py = t // 4
                m = t % 4
                for c in range(3):
                    for kk in range(4):
                        lin[c, py, pl.ds(m * 64 + kk * 16, 16)] = (
                            bi[m, c, py, pl.ds(kk * 16, 16)])
                return carry

            lax.fori_loop(0, 64, body, 0)

        def store(k):
            b, j, q, ph = coords(k)
            return [
                pltpu.async_copy(
                    lins[k % 2],
                    out_hbm.at[b, :, pl.ds(j * 64 + ph * 16, 16),
                               pl.ds(q * 256, 256)],
                    souts[k % 2],
                )
            ]

        in_cps = load(0)
        out_cps = [None] * ipw
        for k in range(ipw):
            for cp in in_cps:
                cp.wait()
            if k + 1 < ipw:
                if k >= 1:
                    for cp in out_cps[k - 1]:
                        cp.wait()
                in_cps = load(k + 1)
            merge(k)
            out_cps[k] = store(k)
        for cp in out_cps[ipw - 2]:
            cp.wait()
        for cp in out_cps[ipw - 1]:
            cp.wait()

    out = unpatch(x)
    return jnp.transpose(out, (0, 2, 3, 1))


# speedup vs baseline: 1.6488x; 1.6488x over previous
import functools

import jax
import jax.numpy as jnp
from jax import lax
from jax.experimental import pallas as pl
from jax.experimental.pallas import tpu as pltpu
from jax.experimental.pallas import tpu_sc as plsc

_NC = 2
_NS = 16
_NW = _NC * _NS
_NB = 3


def kernel(patches):
    batch = patches.shape[0]
    x = jnp.transpose(patches, (0, 1, 2, 5, 3, 4))

    n_items = batch * 8 * 4 * 2
    ipw = n_items // _NW

    mesh = plsc.VectorSubcoreMesh(core_axis_name="c", subcore_axis_name="s")

    @functools.partial(
        pl.kernel,
        mesh=mesh,
        out_type=jax.ShapeDtypeStruct((batch, 3, 512, 512), jnp.float32),
        scratch_types=(
            [pltpu.VMEM((3, 32, 64), jnp.float32) for _ in range(2 * _NB)]
            + [pltpu.VMEM((3, 32, 128), jnp.float32) for _ in range(_NB)]
            + [pltpu.SemaphoreType.DMA for _ in range(2 * _NB)]
        ),
    )
    def unpatch(in_hbm, out_hbm, *scratch):
        b0s = list(scratch[0:_NB])
        b1s = list(scratch[_NB:2 * _NB])
        lins = list(scratch[2 * _NB:3 * _NB])
        sins = list(scratch[3 * _NB:4 * _NB])
        souts = list(scratch[4 * _NB:5 * _NB])

        wid = lax.axis_index("s") * _NC + lax.axis_index("c")
        t0 = wid * ipw

        def coords(k):
            t = t0 + k
            b = t // 64
            r = t % 64
            j = r // 8
            q = r % 8
            return b, j, q // 2, q % 2

        def load(k):
            b, j, i2, ph = coords(k)
            return [
                pltpu.async_copy(
                    in_hbm.at[b, j, 2 * i2 + h, :, pl.ds(ph * 32, 32), :],
                    (b0s if h == 0 else b1s)[k % _NB],
                    sins[k % _NB],
                )
                for h in range(2)
            ]

        def merge(k):
            b0, b1, lin = b0s[k % _NB], b1s[k % _NB], lins[k % _NB]

            def body(py, carry):
                for c in range(3):
                    for kk in range(4):
                        sl = pl.ds(kk * 16, 16)
                        lin[c, py, sl] = b0[c, py, sl]
                        lin[c, py, pl.ds(64 + kk * 16, 16)] = b1[c, py, sl]
                return carry

            lax.fori_loop(0, 32, body, 0)

        def store(k):
            b, j, i2, ph = coords(k)
            return [
                pltpu.async_copy(
                    lins[k % _NB],
                    out_hbm.at[b, :, pl.ds(j * 64 + ph * 32, 32),
                               pl.ds(i2 * 128, 128)],
                    souts[k % _NB],
                )
            ]

        in_cps = [None] * ipw
        out_cps = [None] * ipw
        for k in range(min(_NB - 1, ipw)):
            in_cps[k] = load(k)
        for k in range(ipw):
            for cp in in_cps[k]:
                cp.wait()
            merge(k)
            out_cps[k] = store(k)
            nk = k + _NB - 1
            if nk < ipw:
                if nk - _NB >= 0:
                    for cp in out_cps[nk - _NB]:
                        cp.wait()
                in_cps[nk] = load(nk)
        for k in range(max(0, ipw - _NB), ipw):
            for cp in out_cps[k]:
                cp.wait()

    out = unpatch(x)
    return jnp.transpose(out, (0, 2, 3, 1))


# speedup vs baseline: 1.6730x; 1.0147x over previous
import functools

import jax
import jax.numpy as jnp
from jax import lax
from jax.experimental import pallas as pl
from jax.experimental.pallas import tpu as pltpu
from jax.experimental.pallas import tpu_sc as plsc

_NC = 2
_NS = 16
_NW = _NC * _NS
_NB = 3


def kernel(patches):
    batch = patches.shape[0]
    x = jnp.transpose(patches, (0, 1, 2, 5, 3, 4))

    n_items = batch * 8 * 4 * 2
    ipw = n_items // _NW

    mesh = plsc.VectorSubcoreMesh(core_axis_name="c", subcore_axis_name="s")

    @functools.partial(
        pl.kernel,
        mesh=mesh,
        out_type=jax.ShapeDtypeStruct((batch, 3, 512, 512), jnp.float32),
        scratch_types=(
            [pltpu.VMEM((3, 32, 64), jnp.float32) for _ in range(2 * _NB)]
            + [pltpu.VMEM((3, 32, 128), jnp.float32) for _ in range(_NB)]
            + [pltpu.SemaphoreType.DMA for _ in range(2 * _NB)]
        ),
    )
    def unpatch(in_hbm, out_hbm, *scratch):
        b0s = list(scratch[0:_NB])
        b1s = list(scratch[_NB:2 * _NB])
        lins = list(scratch[2 * _NB:3 * _NB])
        sins = list(scratch[3 * _NB:4 * _NB])
        souts = list(scratch[4 * _NB:5 * _NB])

        wid = lax.axis_index("s") * _NC + lax.axis_index("c")
        t0 = wid * ipw

        def coords(k):
            t = t0 + k
            b = t // 64
            r = t % 64
            j = r // 8
            q = r % 8
            return b, j, q // 2, q % 2

        def load(k):
            b, j, i2, ph = coords(k)
            return [
                pltpu.async_copy(
                    in_hbm.at[b, j, 2 * i2 + h, :, pl.ds(ph * 32, 32), :],
                    (b0s if h == 0 else b1s)[k % _NB],
                    sins[k % _NB],
                )
                for h in range(2)
            ]

        def merge(k):
            b0, b1, lin = b0s[k % _NB], b1s[k % _NB], lins[k % _NB]

            def body(py, carry):
                for c in range(3):
                    for kk in range(4):
                        sl = pl.ds(kk * 16, 16)
                        lin[c, py, sl] = b0[c, py, sl]
                        lin[c, py, pl.ds(64 + kk * 16, 16)] = b1[c, py, sl]
                return carry

            lax.fori_loop(0, 32, body, 0)

        def store(k):
            b, j, i2, ph = coords(k)
            return [
                pltpu.async_copy(
                    lins[k % _NB],
                    out_hbm.at[b, :, pl.ds(j * 64 + ph * 32, 32),
                               pl.ds(i2 * 128, 128)],
                    souts[k % _NB],
                )
            ]

        in_cps = [None] * ipw
        out_cps = [None] * ipw
        for k in range(min(_NB - 1, ipw)):
            in_cps[k] = load(k)
        for k in range(ipw):
            nk = k + _NB - 1
            if nk < ipw:
                in_cps[nk] = load(nk)
            for cp in in_cps[k]:
                cp.wait()
            if k - _NB >= 0:
                for cp in out_cps[k - _NB]:
                    cp.wait()
            merge(k)
            out_cps[k] = store(k)
        for k in range(max(0, ipw - _NB), ipw):
            for cp in out_cps[k]:
                cp.wait()

    out = unpatch(x)
    return jnp.transpose(out, (0, 2, 3, 1))
